# baseline (speedup 1.0000x reference)
import jax
import jax.numpy as jnp
from jax import lax
from jax.experimental import pallas as pl
from jax.experimental.pallas import tpu as pltpu
from jax.experimental.pallas import tpu_sc as plsc

N_NODES = 1000000
MEM = 64
INP = 64
MSG = 100
B = 16384

C = 4096
NB = -(-N_NODES // C)
E = 128
NT = B // E

NC = 2
NS = 16
NW = NC * NS
BPW = B // NW


def _sc_gather_body(lu_hbm, idx_hbm, lu_out, idx_v, lu_v, sem_i, sem_l):
    wid = lax.axis_index("s") * NC + lax.axis_index("c")
    base = wid * BPW
    pltpu.async_copy(idx_hbm.at[pl.ds(base, BPW)], idx_v, sem_i).wait()
    pltpu.async_copy(lu_hbm.at[idx_v], lu_v, sem_l).wait()
    pltpu.sync_copy(lu_v, lu_out.at[pl.ds(base, BPW)])


def _sc_gather(last_update2d, source_nodes):
    mesh = plsc.VectorSubcoreMesh(core_axis_name="c", subcore_axis_name="s")
    return pl.kernel(
        _sc_gather_body,
        out_type=jax.ShapeDtypeStruct((B, 1), jnp.float32),
        mesh=mesh,
        scratch_types=(
            pltpu.VMEM((BPW,), jnp.int32),
            pltpu.VMEM((BPW, 1), jnp.float32),
            pltpu.SemaphoreType.DMA,
            pltpu.SemaphoreType.DMA,
        ),
        compiler_params=pltpu.CompilerParams(use_tc_tiling_on_sc=False),
    )(last_update2d, source_nodes)


def _mega_body(off_ref, mem_ref, colid_ref, colcol_ref, ef_ref, dlt_ref,
               w1a_ref, w1b_ref, w1c_ref, b1_ref, w2_ref, b2_ref,
               wir_ref, wiz_ref, win_ref, whr_ref, whz_ref, whn_ref,
               brz_r_ref, brz_z_ref, bin_ref, bhn_ref,
               out_ref, ef_buf, cc_buf, sem_e, sem_c):
    g = pl.program_id(0)
    base = g * C
    f32 = jnp.float32
    out_ref[...] = mem_ref[...]
    col_ok = (base + lax.broadcasted_iota(jnp.int32, (1, C), 1)) < N_NODES
    x_clean = jnp.where(jnp.broadcast_to(col_ok, (MEM, C)), mem_ref[...], 0.0)
    t0 = off_ref[g] // E
    t1 = (off_ref[g + 1] + (E - 1)) // E

    def chunk(t, carry):
        s0 = pl.multiple_of(t * E, E)
        cp_e = pltpu.make_async_copy(ef_ref.at[pl.ds(t, 1)], ef_buf, sem_e)
        cp_c = pltpu.make_async_copy(colcol_ref.at[pl.ds(s0, E)], cc_buf, sem_c)
        cp_e.start()
        cp_c.start()
        colrel = colid_ref[pl.ds(t, 1), :] - base
        s1 = (lax.broadcasted_iota(jnp.int32, (C, E), 0) == colrel).astype(f32)
        h = jnp.dot(x_clean, s1, preferred_element_type=f32)
        cp_e.wait()
        cp_c.wait()
        ef = ef_buf[0]
        colrel_c = cc_buf[...] - base
        dl = dlt_ref[pl.ds(t, 1), :]
        x1 = (jnp.dot(w1a_ref[...], h, preferred_element_type=f32)
              + jnp.dot(w1b_ref[...], ef, preferred_element_type=f32)
              + w1c_ref[...] * dl + b1_ref[...])
        h1 = jnp.maximum(x1, 0.0)
        msg = jnp.dot(w2_ref[...], h1, preferred_element_type=f32) + b2_ref[...]
        r = jax.nn.sigmoid(jnp.dot(wir_ref[...], msg, preferred_element_type=f32)
                           + jnp.dot(whr_ref[...], h, preferred_element_type=f32)
                           + brz_r_ref[...])
        z = jax.nn.sigmoid(jnp.dot(wiz_ref[...], msg, preferred_element_type=f32)
                           + jnp.dot(whz_ref[...], h, preferred_element_type=f32)
                           + brz_z_ref[...])
        hn = jnp.dot(whn_ref[...], h, preferred_element_type=f32) + bhn_ref[...]
        nn = jnp.tanh(jnp.dot(win_ref[...], msg, preferred_element_type=f32)
                      + bin_ref[...] + r * hn)
        upd = (1.0 - z) * nn + z * h
        iota_ec = lax.broadcasted_iota(jnp.int32, (E, C), 1)
        m_le = (colrel_c <= iota_ec).astype(f32)
        ones_row = jnp.ones((1, E), f32)
        cnt = jnp.dot(ones_row, m_le, preferred_element_type=f32)
        slot = cnt.astype(jnp.int32) - 1
        nneg = jnp.dot(ones_row, (colrel_c < 0).astype(f32),
                       preferred_element_type=f32)
        cnt_prev = jnp.concatenate(
            [jnp.broadcast_to(nneg, (1, 1)), cnt[:, :C - 1]], axis=1)
        hit = cnt > cnt_prev
        sel = (lax.broadcasted_iota(jnp.int32, (E, C), 0) == slot).astype(f32)
        vals = jnp.dot(upd, sel, preferred_element_type=f32)
        out_ref[...] = jnp.where(jnp.broadcast_to(hit, (MEM, C)),
                                 vals, out_ref[...])
        return carry

    lax.fori_loop(t0, t1, chunk, 0)


def _tc_mega(mem_t, colid2, colcol, ef3, dlt2, offsets, ws):
    blk = lambda g: (0, g)
    rep = lambda g: (0, 0)
    rep3 = lambda g: (0, 0, 0)
    return pl.pallas_call(
        _mega_body,
        grid=(NB,),
        in_specs=[
            pl.BlockSpec(memory_space=pltpu.SMEM),
            pl.BlockSpec((MEM, C), blk),
            pl.BlockSpec((NT, E), rep),
            pl.BlockSpec(memory_space=pl.ANY),
            pl.BlockSpec(memory_space=pl.ANY),
            pl.BlockSpec((NT, E), rep),
        ] + [pl.BlockSpec(w.shape, rep) for w in ws],
        out_specs=pl.BlockSpec((MEM, C), blk),
        out_shape=jax.ShapeDtypeStruct((MEM, N_NODES), jnp.float32),
        scratch_shapes=[
            pltpu.VMEM((1, MEM, E), jnp.float32),
            pltpu.VMEM((E, 1), jnp.int32),
            pltpu.SemaphoreType.DMA,
            pltpu.SemaphoreType.DMA,
        ],
    )(offsets, mem_t, colid2, colcol, ef3, dlt2, *ws)


def kernel(source_nodes, edge_times, edge_features, memory, last_update,
           W1, b1, W2, b2, W_ih, W_hh, b_ih, b_hh):
    mem_t = memory.T
    ef_t = edge_features.T
    lu2 = last_update.reshape(N_NODES, 1)

    iota = lax.iota(jnp.int32, B)
    keys_s, perm = lax.sort([source_nodes, iota], num_keys=1, is_stable=True)
    bounds = lax.iota(jnp.int32, NB + 1) * C
    offsets = jnp.searchsorted(keys_s, bounds, side="left",
                               method="compare_all").astype(jnp.int32)

    lu_orig = _sc_gather(lu2, source_nodes)
    delta_orig = edge_times - lu_orig.reshape(B)
    delta_row = jnp.take(delta_orig, perm).reshape(1, B)

    ef_s_t = jnp.take(ef_t, perm, axis=1)
    ef3 = ef_s_t.reshape(MEM, NT, E).transpose(1, 0, 2)
    colid2 = keys_s.reshape(NT, E)
    colcol = keys_s.reshape(B, 1)
    dlt2 = delta_row.reshape(NT, E)

    w1a = W1[:, :MEM]
    w1b = W1[:, MEM:MEM + INP]
    w1c = W1[:, MEM + INP][:, None]
    b1c = b1[:, None]
    b2c = b2[:, None]
    wir = W_ih[0:MEM]
    wiz = W_ih[MEM:2 * MEM]
    win = W_ih[2 * MEM:3 * MEM]
    whr = W_hh[0:MEM]
    whz = W_hh[MEM:2 * MEM]
    whn = W_hh[2 * MEM:3 * MEM]
    brz_r = (b_ih[0:MEM] + b_hh[0:MEM])[:, None]
    brz_z = (b_ih[MEM:2 * MEM] + b_hh[MEM:2 * MEM])[:, None]
    binc = b_ih[2 * MEM:3 * MEM][:, None]
    bhnc = b_hh[2 * MEM:3 * MEM][:, None]
    ws = [w1a, w1b, w1c, b1c, W2, b2c, wir, wiz, win, whr, whz, whn,
          brz_r, brz_z, binc, bhnc]

    out_t = _tc_mega(mem_t, colid2, colcol, ef3, dlt2, offsets, ws)
    return out_t.T


# speedup vs baseline: 1.1963x; 1.1963x over previous
import jax
import jax.numpy as jnp
from jax import lax
from jax.experimental import pallas as pl
from jax.experimental.pallas import tpu as pltpu
from jax.experimental.pallas import tpu_sc as plsc

N_NODES = 1000000
MEM = 64
INP = 64
MSG = 100
B = 16384

C = 4096
NB = -(-N_NODES // C)
E = 128
NT = B // E

NC = 2
NS = 16
NW = NC * NS
BPW = B // NW


def _sc_gather_body(lu_hbm, idx_hbm, lu_out, idx_v, lu_v, sem_i, sem_l):
    wid = lax.axis_index("s") * NC + lax.axis_index("c")
    base = wid * BPW
    pltpu.async_copy(idx_hbm.at[pl.ds(base, BPW)], idx_v, sem_i).wait()
    pltpu.async_copy(lu_hbm.at[idx_v], lu_v, sem_l).wait()
    pltpu.sync_copy(lu_v, lu_out.at[pl.ds(base, BPW)])


def _sc_gather(last_update2d, source_nodes):
    mesh = plsc.VectorSubcoreMesh(core_axis_name="c", subcore_axis_name="s")
    return pl.kernel(
        _sc_gather_body,
        out_type=jax.ShapeDtypeStruct((B, 1), jnp.float32),
        mesh=mesh,
        scratch_types=(
            pltpu.VMEM((BPW,), jnp.int32),
            pltpu.VMEM((BPW, 1), jnp.float32),
            pltpu.SemaphoreType.DMA,
            pltpu.SemaphoreType.DMA,
        ),
        compiler_params=pltpu.CompilerParams(use_tc_tiling_on_sc=False),
    )(last_update2d, source_nodes)


def _mega_body(off_ref, mem_ref, colid_ref, colcol_ref, ef_ref, dlt_ref,
               w1a_ref, w1b_ref, w1c_ref, b1_ref, w2_ref, b2_ref,
               wir_ref, wiz_ref, win_ref, whr_ref, whz_ref, whn_ref,
               brz_r_ref, brz_z_ref, bin_ref, bhn_ref,
               out_ref):
    g = pl.program_id(0)
    base = g * C
    f32 = jnp.float32
    out_ref[...] = mem_ref[...]
    col_ok = (base + lax.broadcasted_iota(jnp.int32, (1, C), 1)) < N_NODES
    x_clean = jnp.where(jnp.broadcast_to(col_ok, (MEM, C)), mem_ref[...], 0.0)
    t0 = off_ref[g] // E
    t1 = (off_ref[g + 1] + (E - 1)) // E

    def chunk(t, carry):
        s0 = pl.multiple_of(t * E, E)
        colrel = colid_ref[pl.ds(t, 1), :] - base
        colrel_c = colcol_ref[pl.ds(s0, E), :] - base
        s1 = (lax.broadcasted_iota(jnp.int32, (C, E), 0) == colrel).astype(f32)
        h = jnp.dot(x_clean, s1, preferred_element_type=f32)
        ef = ef_ref[pl.ds(t, 1)][0]
        dl = dlt_ref[pl.ds(t, 1), :]
        x1 = (jnp.dot(w1a_ref[...], h, preferred_element_type=f32)
              + jnp.dot(w1b_ref[...], ef, preferred_element_type=f32)
              + w1c_ref[...] * dl + b1_ref[...])
        h1 = jnp.maximum(x1, 0.0)
        msg = jnp.dot(w2_ref[...], h1, preferred_element_type=f32) + b2_ref[...]
        r = jax.nn.sigmoid(jnp.dot(wir_ref[...], msg, preferred_element_type=f32)
                           + jnp.dot(whr_ref[...], h, preferred_element_type=f32)
                           + brz_r_ref[...])
        z = jax.nn.sigmoid(jnp.dot(wiz_ref[...], msg, preferred_element_type=f32)
                           + jnp.dot(whz_ref[...], h, preferred_element_type=f32)
                           + brz_z_ref[...])
        hn = jnp.dot(whn_ref[...], h, preferred_element_type=f32) + bhn_ref[...]
        nn = jnp.tanh(jnp.dot(win_ref[...], msg, preferred_element_type=f32)
                      + bin_ref[...] + r * hn)
        upd = (1.0 - z) * nn + z * h
        iota_ec = lax.broadcasted_iota(jnp.int32, (E, C), 1)
        m_le = (colrel_c <= iota_ec).astype(f32)
        ones_row = jnp.ones((1, E), f32)
        cnt = jnp.dot(ones_row, m_le, preferred_element_type=f32)
        slot = cnt.astype(jnp.int32) - 1
        nneg = jnp.dot(ones_row, (colrel_c < 0).astype(f32),
                       preferred_element_type=f32)
        cnt_prev = jnp.concatenate(
            [jnp.broadcast_to(nneg, (1, 1)), cnt[:, :C - 1]], axis=1)
        hit = cnt > cnt_prev
        sel = (lax.broadcasted_iota(jnp.int32, (E, C), 0) == slot).astype(f32)
        vals = jnp.dot(upd, sel, preferred_element_type=f32)
        out_ref[...] = jnp.where(jnp.broadcast_to(hit, (MEM, C)),
                                 vals, out_ref[...])
        return carry

    lax.fori_loop(t0, t1, chunk, 0)


def _tc_mega(mem_t, colid2, colcol, ef3, dlt2, offsets, ws):
    blk = lambda g: (0, g)
    rep = lambda g: (0, 0)
    rep3 = lambda g: (0, 0, 0)
    return pl.pallas_call(
        _mega_body,
        grid=(NB,),
        in_specs=[
            pl.BlockSpec(memory_space=pltpu.SMEM),
            pl.BlockSpec((MEM, C), blk),
            pl.BlockSpec((NT, E), rep),
            pl.BlockSpec((B, 1), rep),
            pl.BlockSpec((NT, MEM, E), rep3),
            pl.BlockSpec((NT, E), rep),
        ] + [pl.BlockSpec(w.shape, rep) for w in ws],
        out_specs=pl.BlockSpec((MEM, C), blk),
        out_shape=jax.ShapeDtypeStruct((MEM, N_NODES), jnp.float32),
    )(offsets, mem_t, colid2, colcol, ef3, dlt2, *ws)


def kernel(source_nodes, edge_times, edge_features, memory, last_update,
           W1, b1, W2, b2, W_ih, W_hh, b_ih, b_hh):
    mem_t = memory.T
    ef_t = edge_features.T
    lu2 = last_update.reshape(N_NODES, 1)

    iota = lax.iota(jnp.int32, B)
    keys_s, perm = lax.sort([source_nodes, iota], num_keys=1, is_stable=True)
    bounds = lax.iota(jnp.int32, NB + 1) * C
    offsets = jnp.searchsorted(keys_s, bounds, side="left",
                               method="compare_all").astype(jnp.int32)

    lu_orig = _sc_gather(lu2, source_nodes)
    delta_orig = edge_times - lu_orig.reshape(B)
    delta_row = jnp.take(delta_orig, perm).reshape(1, B)

    ef_s_t = jnp.take(ef_t, perm, axis=1)
    ef3 = ef_s_t.reshape(MEM, NT, E).transpose(1, 0, 2)
    colid2 = keys_s.reshape(NT, E)
    colcol = keys_s.reshape(B, 1)
    dlt2 = delta_row.reshape(NT, E)

    w1a = W1[:, :MEM]
    w1b = W1[:, MEM:MEM + INP]
    w1c = W1[:, MEM + INP][:, None]
    b1c = b1[:, None]
    b2c = b2[:, None]
    wir = W_ih[0:MEM]
    wiz = W_ih[MEM:2 * MEM]
    win = W_ih[2 * MEM:3 * MEM]
    whr = W_hh[0:MEM]
    whz = W_hh[MEM:2 * MEM]
    whn = W_hh[2 * MEM:3 * MEM]
    brz_r = (b_ih[0:MEM] + b_hh[0:MEM])[:, None]
    brz_z = (b_ih[MEM:2 * MEM] + b_hh[MEM:2 * MEM])[:, None]
    binc = b_ih[2 * MEM:3 * MEM][:, None]
    bhnc = b_hh[2 * MEM:3 * MEM][:, None]
    ws = [w1a, w1b, w1c, b1c, W2, b2c, wir, wiz, win, whr, whz, whn,
          brz_r, brz_z, binc, bhnc]

    out_t = _tc_mega(mem_t, colid2, colcol, ef3, dlt2, offsets, ws)
    return out_t.T
